# Initial kernel scaffold; baseline (speedup 1.0000x reference)
import jax
import jax.numpy as jnp
from jax.experimental import pallas as pl


def kernel(x, attr, edge_index, batch, Wk, bias, ln_g, ln_b, W1, b1, W2, b2, ls):
    raise NotImplementedError("write your pallas kernel here")


# speedup vs baseline: 1.7064x; 1.7064x over previous
import functools

import jax
import jax.numpy as jnp
from jax import lax
from jax.experimental import pallas as pl
from jax.experimental.pallas import tpu as pltpu
from jax.experimental.pallas import tpu_sc as plsc

N = 10000
E = 320000
D = 128
A = 16

NC = 2
NS = 16
NW = NC * NS

C = 128
EW = 10240
EPAD = NW * EW
CHUNKS = EW // C
STRIDE_ROWS = 624
TILE_ROWS = 640
ZR = 40

EB = 1280


def _edge_mm_body(attr_ref, wkt_ref, out_ref):
    out_ref[...] = jnp.dot(attr_ref[...], wkt_ref[...],
                           preferred_element_type=jnp.float32)


def _edge_matmul(attr_pad, wkt):
    grid = EPAD // EB
    return pl.pallas_call(
        _edge_mm_body,
        grid=(grid,),
        in_specs=[
            pl.BlockSpec((EB, A), lambda i: (i, 0)),
            pl.BlockSpec((A, D), lambda i: (0, 0)),
        ],
        out_specs=pl.BlockSpec((EB, D), lambda i: (i, 0)),
        out_shape=jax.ShapeDtypeStruct((EPAD, D), jnp.float32),
    )(attr_pad, wkt)


def _sc_body(src_hbm, dst_hbm, ker_hbm, x_hbm, out_hbm,
             src_v, dst_v, ker_v, rows_v, zero_v, acc_sh, gsem):
    c = lax.axis_index("c")
    s = lax.axis_index("s")
    wid = c * NS + s

    for r in range(ZR):
        for db in range(D // 16):
            zero_v[r, pl.ds(db * 16, 16)] = jnp.zeros((16,), jnp.float32)
    row0 = s * STRIDE_ROWS
    for j in range(TILE_ROWS // ZR):
        pltpu.sync_copy(zero_v, acc_sh.at[pl.ds(row0 + j * ZR, ZR)])
    plsc.subcore_barrier()

    def chunk_body(g, carry):
        base = wid * EW + g * C
        pltpu.sync_copy(src_hbm.at[pl.ds(base, C)], src_v)
        pltpu.sync_copy(dst_hbm.at[pl.ds(base, C)], dst_v)
        pltpu.sync_copy(ker_hbm.at[pl.ds(base, C)], ker_v)
        pltpu.async_copy(x_hbm.at[src_v], rows_v, gsem).wait()

        def mul_body(e, carry2):
            for db in range(D // 16):
                sl = pl.ds(db * 16, 16)
                rows_v[e, sl] = rows_v[e, sl] * ker_v[e, sl]
            return carry2
        lax.fori_loop(0, C, mul_body, 0)

        pltpu.sync_copy(rows_v, acc_sh.at[dst_v], add=True)
        return carry
    lax.fori_loop(0, CHUNKS, chunk_body, 0)
    plsc.subcore_barrier()

    pltpu.sync_copy(acc_sh.at[pl.ds(row0, TILE_ROWS)],
                    out_hbm.at[c, pl.ds(row0, TILE_ROWS)])


def _sc_scatter(src_pad, dst_pad, kernelE, x):
    mesh = plsc.VectorSubcoreMesh(core_axis_name="c", subcore_axis_name="s")
    f = functools.partial(
        pl.kernel,
        out_type=jax.ShapeDtypeStruct((NC, N, D), jnp.float32),
        mesh=mesh,
        scratch_types=[
            pltpu.VMEM((C,), jnp.int32),
            pltpu.VMEM((C,), jnp.int32),
            pltpu.VMEM((C, D), jnp.float32),
            pltpu.VMEM((C, D), jnp.float32),
            pltpu.VMEM((ZR, D), jnp.float32),
            pltpu.VMEM_SHARED((N, D), jnp.float32),
            pltpu.SemaphoreType.DMA,
        ],
    )(_sc_body)
    return f(src_pad, dst_pad, kernelE, x)


def _node_body(parts_ref, x_ref, bias_ref, g_ref, b_ref,
               w1_ref, b1_ref, w2_ref, b2_ref, ls_ref, out_ref):
    h = parts_ref[0] + parts_ref[1] + bias_ref[...]
    mean = jnp.mean(h, axis=-1, keepdims=True)
    cent = h - mean
    var = jnp.mean(cent * cent, axis=-1, keepdims=True)
    h = cent / jnp.sqrt(var + 1e-5) * g_ref[...] + b_ref[...]
    h1 = lax.dot_general(h, w1_ref[...], (((1,), (1,)), ((), ())),
                         preferred_element_type=jnp.float32) + b1_ref[...]
    h1 = jnp.maximum(h1, 0.0)
    h2 = lax.dot_general(h1, w2_ref[...], (((1,), (1,)), ((), ())),
                         preferred_element_type=jnp.float32) + b2_ref[...]
    out_ref[...] = ls_ref[...] * h2 + x_ref[...]


def _node_phase(parts, x, bias, ln_g, ln_b, W1, b1, W2, b2, ls):
    RB = 1000
    grid = N // RB
    H = W1.shape[0]
    return pl.pallas_call(
        _node_body,
        grid=(grid,),
        in_specs=[
            pl.BlockSpec((NC, RB, D), lambda i: (0, i, 0)),
            pl.BlockSpec((RB, D), lambda i: (i, 0)),
            pl.BlockSpec((1, D), lambda i: (0, 0)),
            pl.BlockSpec((1, D), lambda i: (0, 0)),
            pl.BlockSpec((1, D), lambda i: (0, 0)),
            pl.BlockSpec((H, D), lambda i: (0, 0)),
            pl.BlockSpec((1, H), lambda i: (0, 0)),
            pl.BlockSpec((D, H), lambda i: (0, 0)),
            pl.BlockSpec((1, D), lambda i: (0, 0)),
            pl.BlockSpec((1, D), lambda i: (0, 0)),
        ],
        out_specs=pl.BlockSpec((RB, D), lambda i: (i, 0)),
        out_shape=jax.ShapeDtypeStruct((N, D), jnp.float32),
    )(parts, x, bias.reshape(1, D), ln_g.reshape(1, D), ln_b.reshape(1, D),
      W1, b1.reshape(1, H), W2, b2.reshape(1, D), ls.reshape(1, D))


def kernel(x, attr, edge_index, batch, Wk, bias, ln_g, ln_b, W1, b1, W2, b2, ls):
    pad = EPAD - E
    attr_pad = jnp.pad(attr, ((0, pad), (0, 0)))
    src_pad = jnp.pad(edge_index[0], (0, pad))
    dst_pad = jnp.pad(edge_index[1], (0, pad))
    kernelE = _edge_matmul(attr_pad, Wk.T)
    parts = _sc_scatter(src_pad, dst_pad, kernelE, x)
    return _node_phase(parts, x, bias, ln_g, ln_b, W1, b1, W2, b2, ls)


# speedup vs baseline: 2.2736x; 1.3323x over previous
import functools

import jax
import jax.numpy as jnp
from jax import lax
from jax.experimental import pallas as pl
from jax.experimental.pallas import tpu as pltpu
from jax.experimental.pallas import tpu_sc as plsc

N = 10000
E = 320000
D = 128
A = 16

NC = 2
NS = 16
NW = NC * NS

C = 64
EW = 10240
EPAD = NW * EW
CHUNKS = EW // C
assert CHUNKS % 2 == 0
STRIDE_ROWS = 624
TILE_ROWS = 640
ZR = 64

EB = 1280
REAL_BLOCKS = E // EB


def _edge_mm_body(attr_ref, wkt_ref, out_ref):
    i = pl.program_id(0)

    @pl.when(i < REAL_BLOCKS)
    def _():
        out_ref[...] = jnp.dot(attr_ref[...], wkt_ref[...],
                               preferred_element_type=jnp.float32)

    @pl.when(i >= REAL_BLOCKS)
    def _():
        out_ref[...] = jnp.zeros_like(out_ref)


def _edge_matmul(attr, wkt):
    grid = EPAD // EB
    return pl.pallas_call(
        _edge_mm_body,
        grid=(grid,),
        in_specs=[
            pl.BlockSpec((EB, A), lambda i: (jnp.minimum(i, REAL_BLOCKS - 1), 0)),
            pl.BlockSpec((A, D), lambda i: (0, 0)),
        ],
        out_specs=pl.BlockSpec((EB, D), lambda i: (i, 0)),
        out_shape=jax.ShapeDtypeStruct((EPAD, D), jnp.float32),
    )(attr, wkt)


def _sc_body(src_hbm, dst_hbm, ker_hbm, x_hbm, out_hbm,
             src_b, dst_b, ker_b, rows_b, acc_sh,
             sa, sb, sc):
    c = lax.axis_index("c")
    s = lax.axis_index("s")
    wid = c * NS + s
    ebase = wid * EW

    for r in range(ZR):
        for db in range(D // 16):
            rows_b[0, r, pl.ds(db * 16, 16)] = jnp.zeros((16,), jnp.float32)
    row0 = s * STRIDE_ROWS
    for j in range(TILE_ROWS // ZR):
        pltpu.sync_copy(rows_b.at[0], acc_sh.at[pl.ds(row0 + j * ZR, ZR)])
    plsc.subcore_barrier()

    def issue_A(g, b):
        base = ebase + g * C
        pltpu.async_copy(src_hbm.at[pl.ds(base, C)], src_b.at[b], sa[b])
        pltpu.async_copy(ker_hbm.at[pl.ds(base, C)], ker_b.at[b], sa[b])

    def wait_A(b):
        pltpu.make_async_copy(src_hbm.at[pl.ds(0, C)], src_b.at[b], sa[b]).wait()
        pltpu.make_async_copy(ker_hbm.at[pl.ds(0, C)], ker_b.at[b], sa[b]).wait()

    def issue_B(g, b):
        base = ebase + g * C
        pltpu.async_copy(dst_hbm.at[pl.ds(base, C)], dst_b.at[b], sb[b])
        pltpu.async_copy(x_hbm.at[src_b.at[b]], rows_b.at[b], sb[b])

    def wait_B(b):
        pltpu.make_async_copy(dst_hbm.at[pl.ds(0, C)], dst_b.at[b], sb[b]).wait()
        pltpu.make_async_copy(x_hbm.at[src_b.at[b]], rows_b.at[b], sb[b]).wait()

    def issue_S(b):
        pltpu.async_copy(rows_b.at[b], acc_sh.at[dst_b.at[b]], sc[b], add=True)

    def wait_S(b):
        pltpu.make_async_copy(rows_b.at[b], acc_sh.at[dst_b.at[b]], sc[b]).wait()

    def multiply(b):
        def mul_body(e, carry):
            for db in range(D // 16):
                sl = pl.ds(db * 16, 16)
                rows_b[b, e, sl] = rows_b[b, e, sl] * ker_b[b, e, sl]
            return carry
        lax.fori_loop(0, C, mul_body, 0)

    def step(k, b, o):
        @pl.when(k >= 2)
        def _():
            wait_S(b)
        wait_A(b)
        issue_B(k, b)

        @pl.when(k >= 1)
        def _():
            wait_B(o)
            multiply(o)
            issue_S(o)

            @pl.when(k <= CHUNKS - 2)
            def _():
                issue_A(k + 1, o)

    issue_A(0, 0)
    issue_A(1, 1)

    def loop_body(i, carry):
        step(2 * i, 0, 1)
        step(2 * i + 1, 1, 0)
        return carry
    lax.fori_loop(0, CHUNKS // 2, loop_body, 0)

    wait_B(1)
    multiply(1)
    issue_S(1)
    wait_S(0)
    wait_S(1)
    plsc.subcore_barrier()

    pltpu.sync_copy(acc_sh.at[pl.ds(row0, TILE_ROWS)],
                    out_hbm.at[c, pl.ds(row0, TILE_ROWS)])


def _sc_scatter(src_pad, dst_pad, kernelE, x):
    mesh = plsc.VectorSubcoreMesh(core_axis_name="c", subcore_axis_name="s")
    f = functools.partial(
        pl.kernel,
        out_type=jax.ShapeDtypeStruct((NC, N, D), jnp.float32),
        mesh=mesh,
        scratch_types=[
            pltpu.VMEM((2, C), jnp.int32),
            pltpu.VMEM((2, C), jnp.int32),
            pltpu.VMEM((2, C, D), jnp.float32),
            pltpu.VMEM((2, C, D), jnp.float32),
            pltpu.VMEM_SHARED((N, D), jnp.float32),
            [pltpu.SemaphoreType.DMA, pltpu.SemaphoreType.DMA],
            [pltpu.SemaphoreType.DMA, pltpu.SemaphoreType.DMA],
            [pltpu.SemaphoreType.DMA, pltpu.SemaphoreType.DMA],
        ],
    )(_sc_body)
    return f(src_pad, dst_pad, kernelE, x)


def _node_body(parts_ref, x_ref, bias_ref, g_ref, b_ref,
               w1_ref, b1_ref, w2_ref, b2_ref, ls_ref, out_ref):
    h = parts_ref[0] + parts_ref[1] + bias_ref[...]
    mean = jnp.mean(h, axis=-1, keepdims=True)
    cent = h - mean
    var = jnp.mean(cent * cent, axis=-1, keepdims=True)
    h = cent / jnp.sqrt(var + 1e-5) * g_ref[...] + b_ref[...]
    h1 = lax.dot_general(h, w1_ref[...], (((1,), (1,)), ((), ())),
                         preferred_element_type=jnp.float32) + b1_ref[...]
    h1 = jnp.maximum(h1, 0.0)
    h2 = lax.dot_general(h1, w2_ref[...], (((1,), (1,)), ((), ())),
                         preferred_element_type=jnp.float32) + b2_ref[...]
    out_ref[...] = ls_ref[...] * h2 + x_ref[...]


def _node_phase(parts, x, bias, ln_g, ln_b, W1, b1, W2, b2, ls):
    RB = 1000
    grid = N // RB
    H = W1.shape[0]
    return pl.pallas_call(
        _node_body,
        grid=(grid,),
        in_specs=[
            pl.BlockSpec((NC, RB, D), lambda i: (0, i, 0)),
            pl.BlockSpec((RB, D), lambda i: (i, 0)),
            pl.BlockSpec((1, D), lambda i: (0, 0)),
            pl.BlockSpec((1, D), lambda i: (0, 0)),
            pl.BlockSpec((1, D), lambda i: (0, 0)),
            pl.BlockSpec((H, D), lambda i: (0, 0)),
            pl.BlockSpec((1, H), lambda i: (0, 0)),
            pl.BlockSpec((D, H), lambda i: (0, 0)),
            pl.BlockSpec((1, D), lambda i: (0, 0)),
            pl.BlockSpec((1, D), lambda i: (0, 0)),
        ],
        out_specs=pl.BlockSpec((RB, D), lambda i: (i, 0)),
        out_shape=jax.ShapeDtypeStruct((N, D), jnp.float32),
    )(parts, x, bias.reshape(1, D), ln_g.reshape(1, D), ln_b.reshape(1, D),
      W1, b1.reshape(1, H), W2, b2.reshape(1, D), ls.reshape(1, D))


def kernel(x, attr, edge_index, batch, Wk, bias, ln_g, ln_b, W1, b1, W2, b2, ls):
    pad = EPAD - E
    src_pad = jnp.pad(edge_index[0], (0, pad))
    dst_pad = jnp.pad(edge_index[1], (0, pad))
    kernelE = _edge_matmul(attr, Wk.T)
    parts = _sc_scatter(src_pad, dst_pad, kernelE, x)
    return _node_phase(parts, x, bias, ln_g, ln_b, W1, b1, W2, b2, ls)


# speedup vs baseline: 2.3516x; 1.0343x over previous
import functools

import jax
import jax.numpy as jnp
from jax import lax
from jax.experimental import pallas as pl
from jax.experimental.pallas import tpu as pltpu
from jax.experimental.pallas import tpu_sc as plsc

N = 10000
E = 320000
D = 128
A = 16

NC = 2
NS = 16
NW = NC * NS

C = 64
G8 = C // 8
EW = 10240
EPAD = NW * EW
CHUNKS = EW // C
assert CHUNKS % 2 == 0
STRIDE_ROWS = 624
TILE_ROWS = 640
ZR = 64

E8 = E // 8
EPAD8 = EPAD // 8
RB8 = 1024


def _edge_mm_body(a8_ref, m_ref, out_ref):
    out_ref[...] = jnp.dot(a8_ref[...], m_ref[...],
                           preferred_element_type=jnp.float32)


def _edge_matmul(attr8_pad, m):
    grid = EPAD8 // RB8
    return pl.pallas_call(
        _edge_mm_body,
        grid=(grid,),
        in_specs=[
            pl.BlockSpec((RB8, 8 * A), lambda i: (i, 0)),
            pl.BlockSpec((8 * A, 8 * D), lambda i: (0, 0)),
        ],
        out_specs=pl.BlockSpec((RB8, 8 * D), lambda i: (i, 0)),
        out_shape=jax.ShapeDtypeStruct((EPAD8, 8 * D), jnp.float32),
    )(attr8_pad, m)


def _sc_body(src_hbm, dst_hbm, ker_hbm, x_hbm, out_hbm,
             src_b, dst_b, ker_b, rows_b, acc_sh,
             sa, sb, sc):
    c = lax.axis_index("c")
    s = lax.axis_index("s")
    wid = c * NS + s
    ebase = wid * EW
    ebase8 = wid * (EW // 8)

    for r in range(ZR):
        for db in range(D // 16):
            rows_b[0, r, pl.ds(db * 16, 16)] = jnp.zeros((16,), jnp.float32)
    row0 = s * STRIDE_ROWS
    for j in range(TILE_ROWS // ZR):
        pltpu.sync_copy(rows_b.at[0], acc_sh.at[pl.ds(row0 + j * ZR, ZR)])
    plsc.subcore_barrier()

    def issue_A(g, b):
        pltpu.async_copy(src_hbm.at[pl.ds(ebase + g * C, C)], src_b.at[b], sa[b])
        pltpu.async_copy(ker_hbm.at[pl.ds(ebase8 + g * G8, G8)], ker_b.at[b], sa[b])

    def wait_A(b):
        pltpu.make_async_copy(src_hbm.at[pl.ds(0, C)], src_b.at[b], sa[b]).wait()
        pltpu.make_async_copy(ker_hbm.at[pl.ds(0, G8)], ker_b.at[b], sa[b]).wait()

    def issue_B(g, b):
        pltpu.async_copy(dst_hbm.at[pl.ds(ebase + g * C, C)], dst_b.at[b], sb[b])
        pltpu.async_copy(x_hbm.at[src_b.at[b]], rows_b.at[b], sb[b])

    def wait_B(b):
        pltpu.make_async_copy(dst_hbm.at[pl.ds(0, C)], dst_b.at[b], sb[b]).wait()
        pltpu.make_async_copy(x_hbm.at[src_b.at[b]], rows_b.at[b], sb[b]).wait()

    def issue_S(b):
        pltpu.async_copy(rows_b.at[b], acc_sh.at[dst_b.at[b]], sc[b], add=True)

    def wait_S(b):
        pltpu.make_async_copy(rows_b.at[b], acc_sh.at[dst_b.at[b]], sc[b]).wait()

    def multiply(b):
        def mul_body(gr, carry):
            for j in range(8):
                for db in range(D // 16):
                    rows_b[b, gr * 8 + j, pl.ds(db * 16, 16)] = (
                        rows_b[b, gr * 8 + j, pl.ds(db * 16, 16)]
                        * ker_b[b, gr, pl.ds(j * D + db * 16, 16)])
            return carry
        lax.fori_loop(0, G8, mul_body, 0)

    def step(k, b, o):
        @pl.when(k >= 2)
        def _():
            wait_S(b)
        wait_A(b)
        issue_B(k, b)

        @pl.when(k >= 1)
        def _():
            wait_B(o)
            multiply(o)
            issue_S(o)

            @pl.when(k <= CHUNKS - 2)
            def _():
                issue_A(k + 1, o)

    issue_A(0, 0)
    issue_A(1, 1)

    def loop_body(i, carry):
        step(2 * i, 0, 1)
        step(2 * i + 1, 1, 0)
        return carry
    lax.fori_loop(0, CHUNKS // 2, loop_body, 0)

    wait_B(1)
    multiply(1)
    issue_S(1)
    wait_S(0)
    wait_S(1)
    plsc.subcore_barrier()

    pltpu.sync_copy(acc_sh.at[pl.ds(row0, TILE_ROWS)],
                    out_hbm.at[c, pl.ds(row0, TILE_ROWS)])


def _sc_scatter(src_pad, dst_pad, ker8, x):
    mesh = plsc.VectorSubcoreMesh(core_axis_name="c", subcore_axis_name="s")
    f = functools.partial(
        pl.kernel,
        out_type=jax.ShapeDtypeStruct((NC, N, D), jnp.float32),
        mesh=mesh,
        scratch_types=[
            pltpu.VMEM((2, C), jnp.int32),
            pltpu.VMEM((2, C), jnp.int32),
            pltpu.VMEM((2, G8, 8 * D), jnp.float32),
            pltpu.VMEM((2, C, D), jnp.float32),
            pltpu.VMEM_SHARED((N, D), jnp.float32),
            [pltpu.SemaphoreType.DMA, pltpu.SemaphoreType.DMA],
            [pltpu.SemaphoreType.DMA, pltpu.SemaphoreType.DMA],
            [pltpu.SemaphoreType.DMA, pltpu.SemaphoreType.DMA],
        ],
    )(_sc_body)
    return f(src_pad, dst_pad, ker8, x)


def _node_body(parts_ref, x_ref, bias_ref, g_ref, b_ref,
               w1_ref, b1_ref, w2_ref, b2_ref, ls_ref, out_ref):
    h = parts_ref[0] + parts_ref[1] + bias_ref[...]
    mean = jnp.mean(h, axis=-1, keepdims=True)
    cent = h - mean
    var = jnp.mean(cent * cent, axis=-1, keepdims=True)
    h = cent / jnp.sqrt(var + 1e-5) * g_ref[...] + b_ref[...]
    h1 = lax.dot_general(h, w1_ref[...], (((1,), (1,)), ((), ())),
                         preferred_element_type=jnp.float32) + b1_ref[...]
    h1 = jnp.maximum(h1, 0.0)
    h2 = lax.dot_general(h1, w2_ref[...], (((1,), (1,)), ((), ())),
                         preferred_element_type=jnp.float32) + b2_ref[...]
    out_ref[...] = ls_ref[...] * h2 + x_ref[...]


def _node_phase(parts, x, bias, ln_g, ln_b, W1, b1, W2, b2, ls):
    RB = 1000
    grid = N // RB
    H = W1.shape[0]
    return pl.pallas_call(
        _node_body,
        grid=(grid,),
        in_specs=[
            pl.BlockSpec((NC, RB, D), lambda i: (0, i, 0)),
            pl.BlockSpec((RB, D), lambda i: (i, 0)),
            pl.BlockSpec((1, D), lambda i: (0, 0)),
            pl.BlockSpec((1, D), lambda i: (0, 0)),
            pl.BlockSpec((1, D), lambda i: (0, 0)),
            pl.BlockSpec((H, D), lambda i: (0, 0)),
            pl.BlockSpec((1, H), lambda i: (0, 0)),
            pl.BlockSpec((D, H), lambda i: (0, 0)),
            pl.BlockSpec((1, D), lambda i: (0, 0)),
            pl.BlockSpec((1, D), lambda i: (0, 0)),
        ],
        out_specs=pl.BlockSpec((RB, D), lambda i: (i, 0)),
        out_shape=jax.ShapeDtypeStruct((N, D), jnp.float32),
    )(parts, x, bias.reshape(1, D), ln_g.reshape(1, D), ln_b.reshape(1, D),
      W1, b1.reshape(1, H), W2, b2.reshape(1, D), ls.reshape(1, D))


def kernel(x, attr, edge_index, batch, Wk, bias, ln_g, ln_b, W1, b1, W2, b2, ls):
    pad = EPAD - E
    src_pad = jnp.pad(edge_index[0], (0, pad))
    dst_pad = jnp.pad(edge_index[1], (0, pad))
    attr8_pad = jnp.pad(attr.reshape(E8, 8 * A), ((0, EPAD8 - E8), (0, 0)))
    wkt = Wk.T
    m = jnp.zeros((8 * A, 8 * D), jnp.float32)
    for j in range(8):
        m = m.at[j * A:(j + 1) * A, j * D:(j + 1) * D].set(wkt)
    ker8 = _edge_matmul(attr8_pad, m)
    parts = _sc_scatter(src_pad, dst_pad, ker8, x)
    return _node_phase(parts, x, bias, ln_g, ln_b, W1, b1, W2, b2, ls)


# speedup vs baseline: 2.4670x; 1.0491x over previous
import functools

import jax
import jax.numpy as jnp
from jax import lax
from jax.experimental import pallas as pl
from jax.experimental.pallas import tpu as pltpu
from jax.experimental.pallas import tpu_sc as plsc

N = 10000
E = 320000
D = 128
A = 16

NC = 2
NS = 16
NW = NC * NS

C = 128
G8 = C // 8
EW = 10240
EPAD = NW * EW
CHUNKS = EW // C
assert CHUNKS % 2 == 0
STRIDE_ROWS = 624
TILE_ROWS = 640
ZR = 128

E8 = E // 8
EPAD8 = EPAD // 8
RB8 = 1024


def _edge_mm_body(a8_ref, m_ref, out_ref):
    out_ref[...] = jnp.dot(a8_ref[...], m_ref[...],
                           preferred_element_type=jnp.float32)


def _edge_matmul(attr8_pad, m):
    grid = EPAD8 // RB8
    return pl.pallas_call(
        _edge_mm_body,
        grid=(grid,),
        in_specs=[
            pl.BlockSpec((RB8, 8 * A), lambda i: (i, 0)),
            pl.BlockSpec((8 * A, 8 * D), lambda i: (0, 0)),
        ],
        out_specs=pl.BlockSpec((RB8, 8 * D), lambda i: (i, 0)),
        out_shape=jax.ShapeDtypeStruct((EPAD8, 8 * D), jnp.float32),
    )(attr8_pad, m)


def _sc_body(src_hbm, dst_hbm, ker_hbm, x_hbm, out_hbm,
             src_b, dst_b, ker_v, rows_b, acc_sh,
             sa, sb, sc, sk):
    c = lax.axis_index("c")
    s = lax.axis_index("s")
    wid = c * NS + s
    ebase = wid * EW
    ebase8 = wid * (EW // 8)

    for r in range(ZR):
        for db in range(D // 16):
            rows_b[0, r, pl.ds(db * 16, 16)] = jnp.zeros((16,), jnp.float32)
    row0 = s * STRIDE_ROWS
    for j in range(TILE_ROWS // ZR):
        pltpu.sync_copy(rows_b.at[0], acc_sh.at[pl.ds(row0 + j * ZR, ZR)])
    plsc.subcore_barrier()

    def issue_A(g, b):
        pltpu.async_copy(src_hbm.at[pl.ds(ebase + g * C, C)], src_b.at[b], sa[b])

    def wait_A(b):
        pltpu.make_async_copy(src_hbm.at[pl.ds(0, C)], src_b.at[b], sa[b]).wait()

    def issue_K(g):
        pltpu.async_copy(ker_hbm.at[pl.ds(ebase8 + g * G8, G8)], ker_v, sk)

    def wait_K():
        pltpu.make_async_copy(ker_hbm.at[pl.ds(0, G8)], ker_v, sk).wait()

    def issue_B(g, b):
        pltpu.async_copy(dst_hbm.at[pl.ds(ebase + g * C, C)], dst_b.at[b], sb[b])
        pltpu.async_copy(x_hbm.at[src_b.at[b]], rows_b.at[b], sb[b])

    def wait_B(b):
        pltpu.make_async_copy(dst_hbm.at[pl.ds(0, C)], dst_b.at[b], sb[b]).wait()
        pltpu.make_async_copy(x_hbm.at[src_b.at[b]], rows_b.at[b], sb[b]).wait()

    def issue_S(b):
        pltpu.async_copy(rows_b.at[b], acc_sh.at[dst_b.at[b]], sc[b], add=True)

    def wait_S(b):
        pltpu.make_async_copy(rows_b.at[b], acc_sh.at[dst_b.at[b]], sc[b]).wait()

    def multiply(b):
        def mul_body(gr, carry):
            for j in range(8):
                for db in range(D // 16):
                    rows_b[b, gr * 8 + j, pl.ds(db * 16, 16)] = (
                        rows_b[b, gr * 8 + j, pl.ds(db * 16, 16)]
                        * ker_v[gr, pl.ds(j * D + db * 16, 16)])
            return carry
        lax.fori_loop(0, G8, mul_body, 0)

    def step(k, b, o):
        @pl.when(k >= 2)
        def _():
            wait_S(b)
        wait_A(b)
        issue_B(k, b)

        @pl.when(k >= 1)
        def _():
            wait_B(o)
            wait_K()
            multiply(o)
            issue_S(o)
            issue_K(k)

            @pl.when(k <= CHUNKS - 2)
            def _():
                issue_A(k + 1, o)

    issue_A(0, 0)
    issue_A(1, 1)
    issue_K(0)

    def loop_body(i, carry):
        step(2 * i, 0, 1)
        step(2 * i + 1, 1, 0)
        return carry
    lax.fori_loop(0, CHUNKS // 2, loop_body, 0)

    wait_B(1)
    wait_K()
    multiply(1)
    issue_S(1)
    wait_S(0)
    wait_S(1)
    plsc.subcore_barrier()

    pltpu.sync_copy(acc_sh.at[pl.ds(row0, TILE_ROWS)],
                    out_hbm.at[c, pl.ds(row0, TILE_ROWS)])


def _sc_scatter(src_pad, dst_pad, ker8, x):
    mesh = plsc.VectorSubcoreMesh(core_axis_name="c", subcore_axis_name="s")
    f = functools.partial(
        pl.kernel,
        out_type=jax.ShapeDtypeStruct((NC, N, D), jnp.float32),
        mesh=mesh,
        scratch_types=[
            pltpu.VMEM((2, C), jnp.int32),
            pltpu.VMEM((2, C), jnp.int32),
            pltpu.VMEM((G8, 8 * D), jnp.float32),
            pltpu.VMEM((2, C, D), jnp.float32),
            pltpu.VMEM_SHARED((N, D), jnp.float32),
            [pltpu.SemaphoreType.DMA, pltpu.SemaphoreType.DMA],
            [pltpu.SemaphoreType.DMA, pltpu.SemaphoreType.DMA],
            [pltpu.SemaphoreType.DMA, pltpu.SemaphoreType.DMA],
            pltpu.SemaphoreType.DMA,
        ],
    )(_sc_body)
    return f(src_pad, dst_pad, ker8, x)


def _node_body(parts_ref, x_ref, bias_ref, g_ref, b_ref,
               w1_ref, b1_ref, w2_ref, b2_ref, ls_ref, out_ref):
    h = parts_ref[0] + parts_ref[1] + bias_ref[...]
    mean = jnp.mean(h, axis=-1, keepdims=True)
    cent = h - mean
    var = jnp.mean(cent * cent, axis=-1, keepdims=True)
    h = cent / jnp.sqrt(var + 1e-5) * g_ref[...] + b_ref[...]
    h1 = lax.dot_general(h, w1_ref[...], (((1,), (1,)), ((), ())),
                         preferred_element_type=jnp.float32) + b1_ref[...]
    h1 = jnp.maximum(h1, 0.0)
    h2 = lax.dot_general(h1, w2_ref[...], (((1,), (1,)), ((), ())),
                         preferred_element_type=jnp.float32) + b2_ref[...]
    out_ref[...] = ls_ref[...] * h2 + x_ref[...]


def _node_phase(parts, x, bias, ln_g, ln_b, W1, b1, W2, b2, ls):
    RB = 1000
    grid = N // RB
    H = W1.shape[0]
    return pl.pallas_call(
        _node_body,
        grid=(grid,),
        in_specs=[
            pl.BlockSpec((NC, RB, D), lambda i: (0, i, 0)),
            pl.BlockSpec((RB, D), lambda i: (i, 0)),
            pl.BlockSpec((1, D), lambda i: (0, 0)),
            pl.BlockSpec((1, D), lambda i: (0, 0)),
            pl.BlockSpec((1, D), lambda i: (0, 0)),
            pl.BlockSpec((H, D), lambda i: (0, 0)),
            pl.BlockSpec((1, H), lambda i: (0, 0)),
            pl.BlockSpec((D, H), lambda i: (0, 0)),
            pl.BlockSpec((1, D), lambda i: (0, 0)),
            pl.BlockSpec((1, D), lambda i: (0, 0)),
        ],
        out_specs=pl.BlockSpec((RB, D), lambda i: (i, 0)),
        out_shape=jax.ShapeDtypeStruct((N, D), jnp.float32),
    )(parts, x, bias.reshape(1, D), ln_g.reshape(1, D), ln_b.reshape(1, D),
      W1, b1.reshape(1, H), W2, b2.reshape(1, D), ls.reshape(1, D))


def kernel(x, attr, edge_index, batch, Wk, bias, ln_g, ln_b, W1, b1, W2, b2, ls):
    pad = EPAD - E
    src_pad = jnp.pad(edge_index[0], (0, pad))
    dst_pad = jnp.pad(edge_index[1], (0, pad))
    attr8_pad = jnp.pad(attr.reshape(E8, 8 * A), ((0, EPAD8 - E8), (0, 0)))
    wkt = Wk.T
    m = jnp.zeros((8 * A, 8 * D), jnp.float32)
    for j in range(8):
        m = m.at[j * A:(j + 1) * A, j * D:(j + 1) * D].set(wkt)
    ker8 = _edge_matmul(attr8_pad, m)
    parts = _sc_scatter(src_pad, dst_pad, ker8, x)
    return _node_phase(parts, x, bias, ln_g, ln_b, W1, b1, W2, b2, ls)
